# Initial kernel scaffold; baseline (speedup 1.0000x reference)
import jax
import jax.numpy as jnp
from jax.experimental import pallas as pl


def kernel(flat, cu_seqlens, segment_ids, W_enc, b_enc, W_f, b_f, W_dec, b_dec, W_v, b_v):
    raise NotImplementedError("write your pallas kernel here")


# speedup vs baseline: 4.4928x; 4.4928x over previous
import functools

import jax
import jax.numpy as jnp
from jax.experimental import pallas as pl
from jax.experimental.pallas import tpu as pltpu

_TILE = 1024


def _fused_kernel(starts_ref, ends_ref, x_ref, w_enc_ref, b_enc_ref,
                  w_f_ref, b_f_ref, w_dec_ref, b_dec_ref, w_v_ref, b_v_ref,
                  out_ref, val_ref, seg_acc, agent_acc, *, num_rows, num_steps):
    i = pl.program_id(0)

    @pl.when(i == 0)
    def _init():
        seg_acc[...] = jnp.zeros_like(seg_acc)
        agent_acc[...] = jnp.zeros_like(agent_acc)

    b = starts_ref.shape[0]
    tile = x_ref.shape[0]

    row = jax.lax.broadcasted_iota(jnp.int32, (b, tile), 1) + i * tile
    starts = starts_ref[...]
    ends = ends_ref[...]

    row_valid = jax.lax.broadcasted_iota(jnp.int32, (tile, 1), 0) + i * tile < num_rows
    x = jnp.where(row_valid, x_ref[...], 0.0)

    enc = jax.nn.relu(
        jax.lax.dot(x, w_enc_ref[...], preferred_element_type=jnp.float32)
        + b_enc_ref[...]
    )

    seg_oh = ((row >= starts) & (row < ends)).astype(jnp.float32)
    agent_oh = (row == starts).astype(jnp.float32)

    seg_acc[...] += jax.lax.dot(seg_oh, enc, preferred_element_type=jnp.float32)
    agent_acc[...] += jax.lax.dot(agent_oh, enc, preferred_element_type=jnp.float32)

    @pl.when(i == num_steps - 1)
    def _finalize():
        agent = agent_acc[...]
        neigh_sum = seg_acc[...] - agent
        ncount = (ends - starts - 1).astype(jnp.float32)
        denom = jnp.maximum(ncount, 1.0)
        neigh_mean = jnp.where(ncount > 0.0, neigh_sum / denom, 0.0)
        f_in = jnp.concatenate([agent, neigh_mean], axis=-1)
        hidden = jax.nn.relu(
            jax.lax.dot(f_in, w_f_ref[...], preferred_element_type=jnp.float32)
            + b_f_ref[...]
        )
        out_ref[...] = (
            jax.lax.dot(hidden, w_dec_ref[...], preferred_element_type=jnp.float32)
            + b_dec_ref[...]
        )
        val_ref[...] = (
            jax.lax.dot(hidden, w_v_ref[...], preferred_element_type=jnp.float32)
            + b_v_ref[...]
        )


@jax.jit
def kernel(flat, cu_seqlens, segment_ids, W_enc, b_enc, W_f, b_f, W_dec, b_dec, W_v, b_v):
    del segment_ids
    t, e = flat.shape
    bsz = cu_seqlens.shape[0] - 1
    h = W_f.shape[1]
    n_out = W_dec.shape[1]
    num_steps = pl.cdiv(t, _TILE)

    starts = cu_seqlens[:-1].reshape(bsz, 1).astype(jnp.int32)
    ends = cu_seqlens[1:].reshape(bsz, 1).astype(jnp.int32)

    full = lambda shape: pl.BlockSpec(shape, lambda i: (0,) * len(shape))
    out, value = pl.pallas_call(
        functools.partial(_fused_kernel, num_rows=t, num_steps=num_steps),
        grid=(num_steps,),
        in_specs=[
            full((bsz, 1)),
            full((bsz, 1)),
            pl.BlockSpec((_TILE, e), lambda i: (i, 0)),
            full((e, e)),
            full((1, e)),
            full((2 * e, h)),
            full((1, h)),
            full((h, n_out)),
            full((1, n_out)),
            full((h, 1)),
            full((1, 1)),
        ],
        out_specs=[
            full((bsz, n_out)),
            full((bsz, 1)),
        ],
        out_shape=[
            jax.ShapeDtypeStruct((bsz, n_out), jnp.float32),
            jax.ShapeDtypeStruct((bsz, 1), jnp.float32),
        ],
        scratch_shapes=[
            pltpu.VMEM((bsz, e), jnp.float32),
            pltpu.VMEM((bsz, e), jnp.float32),
        ],
    )(
        starts, ends, flat,
        W_enc, b_enc.reshape(1, e),
        W_f, b_f.reshape(1, h),
        W_dec, b_dec.reshape(1, n_out),
        W_v, b_v.reshape(1, 1),
    )
    return (out, value)


# speedup vs baseline: 6.4108x; 1.4269x over previous
import functools

import jax
import jax.numpy as jnp
from jax.experimental import pallas as pl
from jax.experimental.pallas import tpu as pltpu

_TILE = 4096


def _fused_kernel(lo_ref, hi_ref, x_ref, w_enc_ref, b_enc_ref,
                  w_f_ref, b_f_ref, w_dec_ref, b_dec_ref, w_v_ref, b_v_ref,
                  out_ref, val_ref, acc_ref, *, num_rows, num_steps):
    i = pl.program_id(0)

    @pl.when(i == 0)
    def _init():
        acc_ref[...] = jnp.zeros_like(acc_ref)

    b2 = lo_ref.shape[0]
    b = b2 // 2
    tile = x_ref.shape[0]

    def accumulate(x):
        enc = jax.nn.relu(
            jax.lax.dot(x, w_enc_ref[...], preferred_element_type=jnp.float32)
            + b_enc_ref[...]
        )
        row = jax.lax.broadcasted_iota(jnp.int32, (b2, tile), 1) + i * tile
        oh = ((row >= lo_ref[...]) & (row < hi_ref[...])).astype(jnp.float32)
        acc_ref[...] += jax.lax.dot(oh, enc, preferred_element_type=jnp.float32)

    @pl.when(i < num_steps - 1)
    def _main():
        accumulate(x_ref[...])

    @pl.when(i == num_steps - 1)
    def _last():
        valid = (jax.lax.broadcasted_iota(jnp.int32, (tile, 1), 0)
                 + i * tile < num_rows)
        accumulate(jnp.where(valid, x_ref[...], 0.0))

        seg_sum = acc_ref[:b, :]
        agent = acc_ref[b:, :]
        neigh_sum = seg_sum - agent
        ncount = (hi_ref[:b, :] - lo_ref[:b, :] - 1).astype(jnp.float32)
        denom = jnp.maximum(ncount, 1.0)
        neigh_mean = jnp.where(ncount > 0.0, neigh_sum / denom, 0.0)
        f_in = jnp.concatenate([agent, neigh_mean], axis=-1)
        hidden = jax.nn.relu(
            jax.lax.dot(f_in, w_f_ref[...], preferred_element_type=jnp.float32)
            + b_f_ref[...]
        )
        out_ref[...] = (
            jax.lax.dot(hidden, w_dec_ref[...], preferred_element_type=jnp.float32)
            + b_dec_ref[...]
        )
        val_ref[...] = (
            jax.lax.dot(hidden, w_v_ref[...], preferred_element_type=jnp.float32)
            + b_v_ref[...]
        )


@jax.jit
def kernel(flat, cu_seqlens, segment_ids, W_enc, b_enc, W_f, b_f, W_dec, b_dec, W_v, b_v):
    del segment_ids
    t, e = flat.shape
    bsz = cu_seqlens.shape[0] - 1
    h = W_f.shape[1]
    n_out = W_dec.shape[1]
    num_steps = pl.cdiv(t, _TILE)

    starts = cu_seqlens[:-1].reshape(bsz, 1).astype(jnp.int32)
    ends = cu_seqlens[1:].reshape(bsz, 1).astype(jnp.int32)
    lo = jnp.concatenate([starts, starts], axis=0)
    hi = jnp.concatenate([ends, starts + 1], axis=0)

    full = lambda shape: pl.BlockSpec(shape, lambda i: (0,) * len(shape))
    out, value = pl.pallas_call(
        functools.partial(_fused_kernel, num_rows=t, num_steps=num_steps),
        grid=(num_steps,),
        in_specs=[
            full((2 * bsz, 1)),
            full((2 * bsz, 1)),
            pl.BlockSpec((_TILE, e), lambda i: (i, 0)),
            full((e, e)),
            full((1, e)),
            full((2 * e, h)),
            full((1, h)),
            full((h, n_out)),
            full((1, n_out)),
            full((h, 1)),
            full((1, 1)),
        ],
        out_specs=[
            full((bsz, n_out)),
            full((bsz, 1)),
        ],
        out_shape=[
            jax.ShapeDtypeStruct((bsz, n_out), jnp.float32),
            jax.ShapeDtypeStruct((bsz, 1), jnp.float32),
        ],
        scratch_shapes=[
            pltpu.VMEM((2 * bsz, e), jnp.float32),
        ],
    )(
        lo, hi, flat,
        W_enc, b_enc.reshape(1, e),
        W_f, b_f.reshape(1, h),
        W_dec, b_dec.reshape(1, n_out),
        W_v, b_v.reshape(1, 1),
    )
    return (out, value)


# speedup vs baseline: 6.4318x; 1.0033x over previous
import functools

import jax
import jax.numpy as jnp
from jax.experimental import pallas as pl
from jax.experimental.pallas import tpu as pltpu

_TILE = 8192


def _fused_kernel(lo_ref, hi_ref, x_ref, w_enc_ref, b_enc_ref,
                  w_f_ref, b_f_ref, w_dec_ref, b_dec_ref, w_v_ref, b_v_ref,
                  out_ref, val_ref, acc_ref, *, num_rows, num_steps):
    i = pl.program_id(0)

    @pl.when(i == 0)
    def _init():
        acc_ref[...] = jnp.zeros_like(acc_ref)

    b2 = lo_ref.shape[0]
    b = b2 // 2
    tile = x_ref.shape[0]

    def accumulate(x):
        enc = jax.nn.relu(
            jax.lax.dot(x.astype(jnp.bfloat16), w_enc_ref[...],
                        preferred_element_type=jnp.float32)
            + b_enc_ref[...]
        )
        row = jax.lax.broadcasted_iota(jnp.int32, (b2, tile), 1) + i * tile
        oh = ((row >= lo_ref[...]) & (row < hi_ref[...])).astype(jnp.bfloat16)
        acc_ref[...] += jax.lax.dot(oh, enc.astype(jnp.bfloat16),
                                    preferred_element_type=jnp.float32)

    @pl.when(i < num_steps - 1)
    def _main():
        accumulate(x_ref[...])

    @pl.when(i == num_steps - 1)
    def _last():
        valid = (jax.lax.broadcasted_iota(jnp.int32, (tile, 1), 0)
                 + i * tile < num_rows)
        accumulate(jnp.where(valid, x_ref[...], 0.0))

        seg_sum = acc_ref[:b, :]
        agent = acc_ref[b:, :]
        neigh_sum = seg_sum - agent
        ncount = (hi_ref[:b, :] - lo_ref[:b, :] - 1).astype(jnp.float32)
        denom = jnp.maximum(ncount, 1.0)
        neigh_mean = jnp.where(ncount > 0.0, neigh_sum / denom, 0.0)
        f_in = jnp.concatenate([agent, neigh_mean], axis=-1)
        hidden = jax.nn.relu(
            jax.lax.dot(f_in, w_f_ref[...], preferred_element_type=jnp.float32)
            + b_f_ref[...]
        )
        out_ref[...] = (
            jax.lax.dot(hidden, w_dec_ref[...], preferred_element_type=jnp.float32)
            + b_dec_ref[...]
        )
        val_ref[...] = (
            jax.lax.dot(hidden, w_v_ref[...], preferred_element_type=jnp.float32)
            + b_v_ref[...]
        )


@jax.jit
def kernel(flat, cu_seqlens, segment_ids, W_enc, b_enc, W_f, b_f, W_dec, b_dec, W_v, b_v):
    del segment_ids
    t, e = flat.shape
    bsz = cu_seqlens.shape[0] - 1
    h = W_f.shape[1]
    n_out = W_dec.shape[1]
    num_steps = pl.cdiv(t, _TILE)

    starts = cu_seqlens[:-1].reshape(bsz, 1).astype(jnp.int32)
    ends = cu_seqlens[1:].reshape(bsz, 1).astype(jnp.int32)
    lo = jnp.concatenate([starts, starts], axis=0)
    hi = jnp.concatenate([ends, starts + 1], axis=0)

    full = lambda shape: pl.BlockSpec(shape, lambda i: (0,) * len(shape))
    out, value = pl.pallas_call(
        functools.partial(_fused_kernel, num_rows=t, num_steps=num_steps),
        grid=(num_steps,),
        in_specs=[
            full((2 * bsz, 1)),
            full((2 * bsz, 1)),
            pl.BlockSpec((_TILE, e), lambda i: (i, 0)),
            full((e, e)),
            full((1, e)),
            full((2 * e, h)),
            full((1, h)),
            full((h, n_out)),
            full((1, n_out)),
            full((h, 1)),
            full((1, 1)),
        ],
        out_specs=[
            full((bsz, n_out)),
            full((bsz, 1)),
        ],
        out_shape=[
            jax.ShapeDtypeStruct((bsz, n_out), jnp.float32),
            jax.ShapeDtypeStruct((bsz, 1), jnp.float32),
        ],
        scratch_shapes=[
            pltpu.VMEM((2 * bsz, e), jnp.float32),
        ],
    )(
        lo, hi, flat,
        W_enc.astype(jnp.bfloat16), b_enc.reshape(1, e),
        W_f, b_f.reshape(1, h),
        W_dec, b_dec.reshape(1, n_out),
        W_v, b_v.reshape(1, 1),
    )
    return (out, value)


# speedup vs baseline: 6.9160x; 1.0753x over previous
import functools

import jax
import jax.numpy as jnp
from jax.experimental import pallas as pl
from jax.experimental.pallas import tpu as pltpu

_TILE = 8192


def _fused_kernel(cu_ref, x_ref, w_enc_ref, b_enc_ref,
                  w_f_ref, b_f_ref, w_dec_ref, b_dec_ref, w_v_ref, b_v_ref,
                  out_ref, val_ref, acc_ref, *, num_rows, num_steps):
    i = pl.program_id(0)

    @pl.when(i == 0)
    def _init():
        acc_ref[...] = jnp.zeros_like(acc_ref)

    b = cu_ref.shape[0] - 1
    tile = x_ref.shape[0]

    starts = cu_ref[0:b, :]
    ends = cu_ref[1:b + 1, :]
    lo = jnp.concatenate([starts, starts], axis=0)
    hi = jnp.concatenate([ends, starts + 1], axis=0)

    w_enc = w_enc_ref[...].astype(jnp.bfloat16)

    def accumulate(x):
        enc = jax.nn.relu(
            jax.lax.dot(x.astype(jnp.bfloat16), w_enc,
                        preferred_element_type=jnp.float32)
            + b_enc_ref[...]
        )
        row = jax.lax.broadcasted_iota(jnp.int32, (2 * b, tile), 1) + i * tile
        oh = ((row >= lo) & (row < hi)).astype(jnp.bfloat16)
        acc_ref[...] += jax.lax.dot(oh, enc.astype(jnp.bfloat16),
                                    preferred_element_type=jnp.float32)

    @pl.when(i < num_steps - 1)
    def _main():
        accumulate(x_ref[...])

    @pl.when(i == num_steps - 1)
    def _last():
        valid = (jax.lax.broadcasted_iota(jnp.int32, (tile, 1), 0)
                 + i * tile < num_rows)
        accumulate(jnp.where(valid, x_ref[...], 0.0))

        seg_sum = acc_ref[:b, :]
        agent = acc_ref[b:, :]
        neigh_sum = seg_sum - agent
        ncount = (ends - starts - 1).astype(jnp.float32)
        denom = jnp.maximum(ncount, 1.0)
        neigh_mean = jnp.where(ncount > 0.0, neigh_sum / denom, 0.0)
        f_in = jnp.concatenate([agent, neigh_mean], axis=-1)
        hidden = jax.nn.relu(
            jax.lax.dot(f_in, w_f_ref[...], preferred_element_type=jnp.float32)
            + b_f_ref[...]
        )
        out_ref[...] = (
            jax.lax.dot(hidden, w_dec_ref[...], preferred_element_type=jnp.float32)
            + b_dec_ref[...]
        )
        val_ref[...] = (
            jax.lax.dot(hidden, w_v_ref[...], preferred_element_type=jnp.float32)
            + b_v_ref[...]
        )


@jax.jit
def kernel(flat, cu_seqlens, segment_ids, W_enc, b_enc, W_f, b_f, W_dec, b_dec, W_v, b_v):
    del segment_ids
    t, e = flat.shape
    bsz = cu_seqlens.shape[0] - 1
    h = W_f.shape[1]
    n_out = W_dec.shape[1]
    num_steps = pl.cdiv(t, _TILE)

    full = lambda shape: pl.BlockSpec(shape, lambda i: (0,) * len(shape))
    out, value = pl.pallas_call(
        functools.partial(_fused_kernel, num_rows=t, num_steps=num_steps),
        grid=(num_steps,),
        in_specs=[
            full((bsz + 1, 1)),
            pl.BlockSpec((_TILE, e), lambda i: (i, 0)),
            full((e, e)),
            full((1, e)),
            full((2 * e, h)),
            full((1, h)),
            full((h, n_out)),
            full((1, n_out)),
            full((h, 1)),
            full((1, 1)),
        ],
        out_specs=[
            full((bsz, n_out)),
            full((bsz, 1)),
        ],
        out_shape=[
            jax.ShapeDtypeStruct((bsz, n_out), jnp.float32),
            jax.ShapeDtypeStruct((bsz, 1), jnp.float32),
        ],
        scratch_shapes=[
            pltpu.VMEM((2 * bsz, e), jnp.float32),
        ],
    )(
        cu_seqlens.reshape(bsz + 1, 1), flat,
        W_enc, b_enc.reshape(1, e),
        W_f, b_f.reshape(1, h),
        W_dec, b_dec.reshape(1, n_out),
        W_v, b_v.reshape(1, 1),
    )
    return (out, value)


# speedup vs baseline: 16.3252x; 2.3605x over previous
import functools

import jax
import jax.numpy as jnp
from jax.experimental import pallas as pl
from jax.experimental.pallas import tpu as pltpu

_TILE = 8192


def _fused_kernel(cu_ref, xt_ref, w_enc_ref, b_enc_ref,
                  w_f_ref, b_f_ref, w_dect_ref, b_dec_ref, w_vt_ref, b_v_ref,
                  out_ref, val_ref, acc_ref, *, num_rows, num_steps):
    i = pl.program_id(0)

    @pl.when(i == 0)
    def _init():
        acc_ref[...] = jnp.zeros_like(acc_ref)

    b = cu_ref.shape[1] - 1
    e = xt_ref.shape[0]
    tile = xt_ref.shape[1]

    starts = cu_ref[:, 0:b]
    ends = cu_ref[:, 1:b + 1]
    lo = jnp.concatenate([starts, starts], axis=1)
    hi = jnp.concatenate([ends, starts + 1], axis=1)

    w_enc = w_enc_ref[...].astype(jnp.bfloat16)
    b_enc_col = b_enc_ref[...].reshape(e, 1)

    def accumulate(xt):
        enc_t = jax.nn.relu(
            jax.lax.dot_general(w_enc, xt.astype(jnp.bfloat16),
                                (((0,), (0,)), ((), ())),
                                preferred_element_type=jnp.float32)
            + b_enc_col
        )
        col = jax.lax.broadcasted_iota(jnp.int32, (tile, 2 * b), 0) + i * tile
        oh = ((col >= lo) & (col < hi)).astype(jnp.bfloat16)
        acc_ref[...] += jax.lax.dot(enc_t.astype(jnp.bfloat16), oh,
                                    preferred_element_type=jnp.float32)

    @pl.when(i < num_steps - 1)
    def _main():
        accumulate(xt_ref[...])

    @pl.when(i == num_steps - 1)
    def _last():
        valid = (jax.lax.broadcasted_iota(jnp.int32, (1, tile), 1)
                 + i * tile < num_rows)
        accumulate(jnp.where(valid, xt_ref[...], 0.0))

        seg_sum_t = acc_ref[:, 0:b]
        agent_t = acc_ref[:, b:2 * b]
        neigh_sum_t = seg_sum_t - agent_t
        ncount = (ends - starts - 1).astype(jnp.float32)
        denom = jnp.maximum(ncount, 1.0)
        neigh_mean_t = jnp.where(ncount > 0.0, neigh_sum_t / denom, 0.0)
        f_in_t = jnp.concatenate([agent_t, neigh_mean_t], axis=0)
        f_in = f_in_t.T
        hidden = jax.nn.relu(
            jax.lax.dot(f_in, w_f_ref[...], preferred_element_type=jnp.float32)
            + b_f_ref[...]
        )
        out_ref[...] = (
            jax.lax.dot_general(hidden, w_dect_ref[...],
                                (((1,), (1,)), ((), ())),
                                preferred_element_type=jnp.float32)
            + b_dec_ref[...]
        )
        val_ref[...] = (
            jnp.sum(hidden * w_vt_ref[...], axis=1, keepdims=True)
            + b_v_ref[...]
        )


@jax.jit
def kernel(flat, cu_seqlens, segment_ids, W_enc, b_enc, W_f, b_f, W_dec, b_dec, W_v, b_v):
    del segment_ids
    t, e = flat.shape
    bsz = cu_seqlens.shape[0] - 1
    h = W_f.shape[1]
    n_out = W_dec.shape[1]
    num_steps = pl.cdiv(t, _TILE)

    full = lambda shape: pl.BlockSpec(shape, lambda i: (0,) * len(shape))
    out, value = pl.pallas_call(
        functools.partial(_fused_kernel, num_rows=t, num_steps=num_steps),
        grid=(num_steps,),
        in_specs=[
            full((1, bsz + 1)),
            pl.BlockSpec((e, _TILE), lambda i: (0, i)),
            full((e, e)),
            full((1, e)),
            full((2 * e, h)),
            full((1, h)),
            full((n_out, h)),
            full((1, n_out)),
            full((1, h)),
            full((1, 1)),
        ],
        out_specs=[
            full((bsz, n_out)),
            full((bsz, 1)),
        ],
        out_shape=[
            jax.ShapeDtypeStruct((bsz, n_out), jnp.float32),
            jax.ShapeDtypeStruct((bsz, 1), jnp.float32),
        ],
        scratch_shapes=[
            pltpu.VMEM((e, 2 * bsz), jnp.float32),
        ],
    )(
        cu_seqlens.reshape(1, bsz + 1), flat.T,
        W_enc, b_enc.reshape(1, e),
        W_f, b_f.reshape(1, h),
        W_dec.T, b_dec.reshape(1, n_out),
        W_v.T, b_v.reshape(1, 1),
    )
    return (out, value)


# speedup vs baseline: 18.0848x; 1.1078x over previous
import functools

import jax
import jax.numpy as jnp
from jax.experimental import pallas as pl
from jax.experimental.pallas import tpu as pltpu

_TILE = 8192


def _fused_kernel(cu_ref, xt_ref, w_enc_ref, b_enc_ref,
                  w_f_ref, b_f_ref, w_dect_ref, b_dec_ref, w_vt_ref, b_v_ref,
                  out_ref, val_ref, acc_ref, *, num_rows, num_steps):
    i = pl.program_id(0)

    @pl.when(i == 0)
    def _init():
        acc_ref[...] = jnp.zeros_like(acc_ref)

    b = cu_ref.shape[1] - 1
    e = xt_ref.shape[0]
    tile = xt_ref.shape[1]

    starts = cu_ref[:, 0:b]
    ends = cu_ref[:, 1:b + 1]
    lo = jnp.concatenate([starts, starts], axis=1)
    hi = jnp.concatenate([ends, starts + 1], axis=1)

    w_enc = w_enc_ref[...].astype(jnp.bfloat16)
    b_enc_col = b_enc_ref[...].reshape(e, 1)

    def accumulate(xt):
        enc_t = jax.nn.relu(
            jax.lax.dot_general(w_enc, xt.astype(jnp.bfloat16),
                                (((0,), (0,)), ((), ())),
                                preferred_element_type=jnp.float32)
            + b_enc_col
        )
        col = jax.lax.broadcasted_iota(jnp.int32, (tile, 2 * b), 0) + i * tile
        oh = ((col >= lo) & (col < hi)).astype(jnp.bfloat16)
        acc_ref[...] += jax.lax.dot(enc_t.astype(jnp.bfloat16), oh,
                                    preferred_element_type=jnp.float32)

    @pl.when(i < num_steps - 1)
    def _main():
        accumulate(xt_ref[...])

    @pl.when(i == num_steps - 1)
    def _last():
        valid = (jax.lax.broadcasted_iota(jnp.int32, (1, tile), 1)
                 + i * tile < num_rows)
        accumulate(jnp.where(valid, xt_ref[...], 0.0))

        seg_sum_t = acc_ref[:, 0:b]
        agent_t = acc_ref[:, b:2 * b]
        neigh_sum_t = seg_sum_t - agent_t
        ncount = (ends - starts - 1).astype(jnp.float32)
        denom = jnp.maximum(ncount, 1.0)
        neigh_mean_t = jnp.where(ncount > 0.0, neigh_sum_t / denom, 0.0)
        f_in_t = jnp.concatenate([agent_t, neigh_mean_t], axis=0)
        hidden_t = jax.nn.relu(
            jax.lax.dot_general(w_f_ref[...], f_in_t,
                                (((0,), (0,)), ((), ())),
                                preferred_element_type=jnp.float32)
            + b_f_ref[...].reshape(w_f_ref.shape[1], 1)
        )
        out_ref[...] = (
            jax.lax.dot_general(hidden_t, w_dect_ref[...],
                                (((0,), (1,)), ((), ())),
                                preferred_element_type=jnp.float32)
            + b_dec_ref[...]
        )
        val_ref[...] = (
            jnp.sum(w_vt_ref[...].reshape(hidden_t.shape[0], 1) * hidden_t,
                    axis=0, keepdims=True)
            + b_v_ref[...]
        )


@jax.jit
def kernel(flat, cu_seqlens, segment_ids, W_enc, b_enc, W_f, b_f, W_dec, b_dec, W_v, b_v):
    del segment_ids
    t, e = flat.shape
    bsz = cu_seqlens.shape[0] - 1
    h = W_f.shape[1]
    n_out = W_dec.shape[1]
    num_steps = pl.cdiv(t, _TILE)

    full = lambda shape: pl.BlockSpec(shape, lambda i: (0,) * len(shape))
    out_t, value_t = pl.pallas_call(
        functools.partial(_fused_kernel, num_rows=t, num_steps=num_steps),
        grid=(num_steps,),
        in_specs=[
            full((1, bsz + 1)),
            pl.BlockSpec((e, _TILE), lambda i: (0, i)),
            full((e, e)),
            full((1, e)),
            full((2 * e, h)),
            full((1, h)),
            full((n_out, h)),
            full((1, n_out)),
            full((1, h)),
            full((1, 1)),
        ],
        out_specs=[
            full((bsz, n_out)),
            full((1, bsz)),
        ],
        out_shape=[
            jax.ShapeDtypeStruct((bsz, n_out), jnp.float32),
            jax.ShapeDtypeStruct((1, bsz), jnp.float32),
        ],
        scratch_shapes=[
            pltpu.VMEM((e, 2 * bsz), jnp.float32),
        ],
    )(
        cu_seqlens.reshape(1, bsz + 1), flat.T,
        W_enc, b_enc.reshape(1, e),
        W_f, b_f.reshape(1, h),
        W_dec.T, b_dec.reshape(1, n_out),
        W_v.T, b_v.reshape(1, 1),
    )
    return (out_t, value_t.T)


# speedup vs baseline: 23.3673x; 1.2921x over previous
import functools

import jax
import jax.numpy as jnp
import numpy as np
from jax.experimental import pallas as pl
from jax.experimental.pallas import tpu as pltpu

_TILE = 8192


def _fused_kernel(cu_ref, col_ref, xt_ref, w_enc_ref, b_enc_ref,
                  w_f_ref, b_f_ref, w_dect_ref, b_dec_ref, w_vt_ref, b_v_ref,
                  out_ref, val_ref, acc_ref, *, num_rows, num_steps):
    i = pl.program_id(0)

    @pl.when(i == 0)
    def _init():
        acc_ref[...] = jnp.zeros_like(acc_ref)

    b = cu_ref.shape[1] - 1
    tile = xt_ref.shape[1]

    starts_row = cu_ref[:, 0:b]
    ends_row = cu_ref[:, 1:b + 1]
    cu_col = cu_ref[...].reshape(b + 1, 1)
    starts = cu_col[0:b, :]
    ends = cu_col[1:b + 1, :]
    lo = jnp.concatenate([starts, starts], axis=0) - i * tile
    hi = jnp.concatenate([ends, starts + 1], axis=0) - i * tile

    w_enc = w_enc_ref[...].astype(jnp.bfloat16)
    b_enc_col = b_enc_ref[...].reshape(b_enc_ref.shape[1], 1).astype(jnp.bfloat16)
    col = col_ref[...]

    def accumulate(xt):
        z = jax.lax.dot_general(w_enc, xt.astype(jnp.bfloat16),
                                (((0,), (0,)), ((), ())),
                                preferred_element_type=jnp.float32)
        enc_t = jax.nn.relu(z.astype(jnp.bfloat16) + b_enc_col)
        oh = ((col >= lo) & (col < hi)).astype(jnp.bfloat16)
        acc_ref[...] += jax.lax.dot_general(
            enc_t, oh, (((1,), (1,)), ((), ())),
            preferred_element_type=jnp.float32)

    @pl.when(i < num_steps - 1)
    def _main():
        accumulate(xt_ref[...])

    @pl.when(i == num_steps - 1)
    def _last():
        valid = col + i * tile < num_rows
        accumulate(jnp.where(valid, xt_ref[...], 0.0))

        seg_sum_t = acc_ref[:, 0:b]
        agent_t = acc_ref[:, b:2 * b]
        neigh_sum_t = seg_sum_t - agent_t
        ncount = (ends_row - starts_row - 1).astype(jnp.float32)
        denom = jnp.maximum(ncount, 1.0)
        neigh_mean_t = jnp.where(ncount > 0.0, neigh_sum_t / denom, 0.0)
        f_in_t = jnp.concatenate([agent_t, neigh_mean_t], axis=0)
        hidden_t = jax.nn.relu(
            jax.lax.dot_general(w_f_ref[...], f_in_t,
                                (((0,), (0,)), ((), ())),
                                preferred_element_type=jnp.float32)
            + b_f_ref[...].reshape(w_f_ref.shape[1], 1)
        )
        out_ref[...] = (
            jax.lax.dot_general(hidden_t, w_dect_ref[...],
                                (((0,), (1,)), ((), ())),
                                preferred_element_type=jnp.float32)
            + b_dec_ref[...]
        )
        val_ref[...] = (
            jnp.sum(w_vt_ref[...].reshape(hidden_t.shape[0], 1) * hidden_t,
                    axis=0, keepdims=True)
            + b_v_ref[...]
        )


@jax.jit
def kernel(flat, cu_seqlens, segment_ids, W_enc, b_enc, W_f, b_f, W_dec, b_dec, W_v, b_v):
    del segment_ids
    t, e = flat.shape
    bsz = cu_seqlens.shape[0] - 1
    h = W_f.shape[1]
    n_out = W_dec.shape[1]
    num_steps = pl.cdiv(t, _TILE)
    col = jnp.asarray(np.arange(_TILE, dtype=np.int32).reshape(1, _TILE))

    full = lambda shape: pl.BlockSpec(shape, lambda i: (0,) * len(shape))
    out, value_t = pl.pallas_call(
        functools.partial(_fused_kernel, num_rows=t, num_steps=num_steps),
        grid=(num_steps,),
        in_specs=[
            full((1, bsz + 1)),
            full((1, _TILE)),
            pl.BlockSpec((e, _TILE), lambda i: (0, i)),
            full((e, e)),
            full((1, e)),
            full((2 * e, h)),
            full((1, h)),
            full((n_out, h)),
            full((1, n_out)),
            full((1, h)),
            full((1, 1)),
        ],
        out_specs=[
            full((bsz, n_out)),
            full((1, bsz)),
        ],
        out_shape=[
            jax.ShapeDtypeStruct((bsz, n_out), jnp.float32),
            jax.ShapeDtypeStruct((1, bsz), jnp.float32),
        ],
        scratch_shapes=[
            pltpu.VMEM((e, 2 * bsz), jnp.float32),
        ],
    )(
        cu_seqlens.reshape(1, bsz + 1), col, flat.T,
        W_enc, b_enc.reshape(1, e),
        W_f, b_f.reshape(1, h),
        W_dec.T, b_dec.reshape(1, n_out),
        W_v.T, b_v.reshape(1, 1),
    )
    return (out, value_t.T)


# speedup vs baseline: 25.2756x; 1.0817x over previous
import functools

import jax
import jax.numpy as jnp
import numpy as np
from jax.experimental import pallas as pl
from jax.experimental.pallas import tpu as pltpu

_TILE = 8320


def _fused_kernel(cu_ref, col_ref, xt_ref, w_enc_ref, b_enc_ref,
                  w_f_ref, b_f_ref, w_dect_ref, b_dec_ref, w_vt_ref, b_v_ref,
                  out_ref, val_ref, acc_ref, *, num_rows, num_steps):
    i = pl.program_id(0)

    @pl.when(i == 0)
    def _init():
        acc_ref[...] = jnp.zeros_like(acc_ref)

    b = cu_ref.shape[1] - 1
    tile = xt_ref.shape[1]

    starts_row = cu_ref[:, 0:b]
    ends_row = cu_ref[:, 1:b + 1]
    cu_col = cu_ref[...].reshape(b + 1, 1)
    starts = cu_col[0:b, :]
    ends = cu_col[1:b + 1, :]
    lo = jnp.concatenate([starts, starts], axis=0) - i * tile
    hi = jnp.concatenate([ends, starts + 1], axis=0) - i * tile

    w_enc = w_enc_ref[...].astype(jnp.bfloat16)
    b_enc_col = b_enc_ref[...].reshape(b_enc_ref.shape[1], 1).astype(jnp.bfloat16)
    col = col_ref[...]

    def accumulate(xt):
        z = jax.lax.dot_general(w_enc, xt.astype(jnp.bfloat16),
                                (((0,), (0,)), ((), ())),
                                preferred_element_type=jnp.float32)
        enc_t = jax.nn.relu(z.astype(jnp.bfloat16) + b_enc_col)
        oh = ((col >= lo) & (col < hi)).astype(jnp.bfloat16)
        acc_ref[...] += jax.lax.dot_general(
            enc_t, oh, (((1,), (1,)), ((), ())),
            preferred_element_type=jnp.float32)

    @pl.when(i < num_steps - 1)
    def _main():
        accumulate(xt_ref[...])

    @pl.when(i == num_steps - 1)
    def _last():
        valid = col + i * tile < num_rows
        accumulate(jnp.where(valid, xt_ref[...], 0.0))

        seg_sum_t = acc_ref[:, 0:b]
        agent_t = acc_ref[:, b:2 * b]
        neigh_sum_t = seg_sum_t - agent_t
        ncount = (ends_row - starts_row - 1).astype(jnp.float32)
        denom = jnp.maximum(ncount, 1.0)
        neigh_mean_t = jnp.where(ncount > 0.0, neigh_sum_t / denom, 0.0)
        f_in_t = jnp.concatenate([agent_t, neigh_mean_t], axis=0)
        hidden_t = jax.nn.relu(
            jax.lax.dot_general(w_f_ref[...], f_in_t,
                                (((0,), (0,)), ((), ())),
                                preferred_element_type=jnp.float32)
            + b_f_ref[...].reshape(w_f_ref.shape[1], 1)
        )
        out_ref[...] = (
            jax.lax.dot_general(hidden_t, w_dect_ref[...],
                                (((0,), (1,)), ((), ())),
                                preferred_element_type=jnp.float32)
            + b_dec_ref[...]
        )
        val_ref[...] = (
            jnp.sum(w_vt_ref[...].reshape(hidden_t.shape[0], 1) * hidden_t,
                    axis=0, keepdims=True)
            + b_v_ref[...]
        )


@jax.jit
def kernel(flat, cu_seqlens, segment_ids, W_enc, b_enc, W_f, b_f, W_dec, b_dec, W_v, b_v):
    del segment_ids
    t, e = flat.shape
    bsz = cu_seqlens.shape[0] - 1
    h = W_f.shape[1]
    n_out = W_dec.shape[1]
    num_steps = pl.cdiv(t, _TILE)
    col = jnp.asarray(np.arange(_TILE, dtype=np.int32).reshape(1, _TILE))

    full = lambda shape: pl.BlockSpec(shape, lambda i: (0,) * len(shape))
    out, value_t = pl.pallas_call(
        functools.partial(_fused_kernel, num_rows=t, num_steps=num_steps),
        grid=(num_steps,),
        in_specs=[
            full((1, bsz + 1)),
            full((1, _TILE)),
            pl.BlockSpec((e, _TILE), lambda i: (0, i)),
            full((e, e)),
            full((1, e)),
            full((2 * e, h)),
            full((1, h)),
            full((n_out, h)),
            full((1, n_out)),
            full((1, h)),
            full((1, 1)),
        ],
        out_specs=[
            full((bsz, n_out)),
            full((1, bsz)),
        ],
        out_shape=[
            jax.ShapeDtypeStruct((bsz, n_out), jnp.float32),
            jax.ShapeDtypeStruct((1, bsz), jnp.float32),
        ],
        scratch_shapes=[
            pltpu.VMEM((e, 2 * bsz), jnp.float32),
        ],
    )(
        cu_seqlens.reshape(1, bsz + 1), col, flat.T,
        W_enc, b_enc.reshape(1, e),
        W_f, b_f.reshape(1, h),
        W_dec.T, b_dec.reshape(1, n_out),
        W_v.T, b_v.reshape(1, 1),
    )
    return (out, value_t.T)
